# baseline (speedup 1.0000x reference)
import functools

import jax
import jax.numpy as jnp
from jax import lax
from jax.experimental import pallas as pl
from jax.experimental.pallas import tpu as pltpu
from jax.experimental.pallas import tpu_sc as plsc

N_NODES = 10000
N_EDGES = 320000
DIM = 128
HDIM = 64

NC, NS = 2, 16
NW = NC * NS
E_PER_W = N_EDGES // NW
CHUNK_E = 40
CHUNK = 2 * CHUNK_E
N_PHASES = 2
E_PHASE = E_PER_W // N_PHASES
N_CHUNKS = E_PHASE // CHUNK_E
STAGE_ROWS = 1000


def _round_bf16_bits(u):
    return (u + 0x7FFF + ((u >> 16) & 1)) & jnp.uint32(0xFFFF0000)


def _pack_cols(t):
    lo = lax.bitcast_convert_type(t[:, :HDIM], jnp.uint32)
    hi = lax.bitcast_convert_type(t[:, HDIM:], jnp.uint32)
    return (_round_bf16_bits(lo) >> 16) | _round_bf16_bits(hi)


def _table_kernel(x_ref, w_ref, b_ref, t_ref):
    x = x_ref[...]
    ts = jnp.dot(x, w_ref[0], preferred_element_type=jnp.float32) + b_ref[0]
    te = jnp.dot(x, w_ref[1], preferred_element_type=jnp.float32)
    t_ref[...] = jnp.concatenate([_pack_cols(ts), _pack_cols(te)], axis=1)


def _build_table(x, W1, b1):
    wr = W1.reshape(2, DIM, DIM)
    blk = 1000
    return pl.pallas_call(
        _table_kernel,
        grid=(N_NODES // blk,),
        in_specs=[
            pl.BlockSpec((blk, DIM), lambda i: (i, 0)),
            pl.BlockSpec((2, DIM, DIM), lambda i: (0, 0, 0)),
            pl.BlockSpec((1, DIM), lambda i: (0, 0)),
        ],
        out_specs=pl.BlockSpec((blk, DIM), lambda i: (i, 0)),
        out_shape=jax.ShapeDtypeStruct((N_NODES, DIM), jnp.uint32),
    )(x, wr, b1.reshape(1, DIM))


def _sc_gather(table, idx):
    mesh = plsc.VectorSubcoreMesh(
        core_axis_name="c", subcore_axis_name="s", num_cores=NC, num_subcores=NS
    )

    @functools.partial(
        pl.kernel,
        out_type=jax.ShapeDtypeStruct((N_EDGES, DIM), jnp.uint32),
        mesh=mesh,
        scratch_types=[
            pltpu.VMEM((2 * E_PHASE,), jnp.int32),
            pltpu.VMEM((2, CHUNK, DIM), jnp.uint32),
            pltpu.VMEM((2, CHUNK_E, DIM), jnp.uint32),
            pltpu.VMEM_SHARED((N_NODES, DIM), jnp.uint32),
            pltpu.SemaphoreType.DMA,
            pltpu.SemaphoreType.DMA,
            pltpu.SemaphoreType.DMA,
            pltpu.SemaphoreType.DMA,
            pltpu.SemaphoreType.DMA,
        ],
    )
    def k(t_hbm, i_hbm, o_hbm, idx_v, rows_v, cbuf, t_sp, isem, gsem0, gsem1,
          osem0, osem1):
        cid = lax.axis_index("c")
        sid = lax.axis_index("s")
        wid = sid * NC + cid
        ebase = wid * E_PER_W
        gsems = (gsem0, gsem1)
        osems = (osem0, osem1)

        @pl.when(sid < N_NODES // STAGE_ROWS)
        def _():
            pltpu.sync_copy(
                t_hbm.at[pl.ds(sid * STAGE_ROWS, STAGE_ROWS)],
                t_sp.at[pl.ds(sid * STAGE_ROWS, STAGE_ROWS)],
            )

        plsc.subcore_barrier()

        def compact(b):
            @pl.loop(0, CHUNK_E)
            def _(j):
                for v in range(HDIM // 16):
                    s = pl.ds(16 * v, 16)
                    h = pl.ds(HDIM + 16 * v, 16)
                    cbuf[b, j, s] = rows_v[b, 2 * j, s]
                    cbuf[b, j, h] = rows_v[b, 2 * j + 1, h]

        for p in range(N_PHASES):
            pbase = ebase + p * E_PHASE

            pltpu.async_copy(
                i_hbm.at[pl.ds(2 * pbase, 2 * E_PHASE)], idx_v, isem
            ).wait()

            def gather_copy(ci, b):
                return pltpu.make_async_copy(
                    t_sp.at[idx_v.at[pl.ds(ci * CHUNK, CHUNK)]], rows_v.at[b],
                    gsems[b],
                )

            def out_copy(ci, b):
                return pltpu.make_async_copy(
                    cbuf.at[b],
                    o_hbm.at[pl.ds(pbase + ci * CHUNK_E, CHUNK_E)],
                    osems[b],
                )

            gather_copy(0, 0).start()
            n_paired = N_CHUNKS - (N_CHUNKS % 2)

            @pl.loop(0, n_paired, step=2)
            def _(i0):
                gather_copy(i0, 0).wait()
                compact(0)
                out_copy(i0, 0).start()

                @pl.when(i0 > 0)
                def _():
                    out_copy(i0 - 1, 1).wait()

                gather_copy(i0 + 1, 1).start()

                gather_copy(i0 + 1, 1).wait()
                compact(1)
                out_copy(i0 + 1, 1).start()

                @pl.when(i0 + 2 < N_CHUNKS)
                def _():
                    out_copy(i0, 0).wait()
                    gather_copy(i0 + 2, 0).start()

            if N_CHUNKS % 2:
                gather_copy(N_CHUNKS - 1, 0).wait()
                compact(0)
                out_copy(N_CHUNKS - 1, 0).start()
                out_copy(N_CHUNKS - 2, 1).wait()
                out_copy(N_CHUNKS - 1, 0).wait()
            else:
                out_copy(N_CHUNKS - 2, 0).wait()
                out_copy(N_CHUNKS - 1, 1).wait()

    return k(table, idx)


def _unpack(u):
    lo = lax.bitcast_convert_type(u << 16, jnp.float32)
    hi = lax.bitcast_convert_type(u & jnp.uint32(0xFFFF0000), jnp.float32)
    return lo, hi


def _mlp_kernel(g_ref, w_ref, bias_ref, o_ref):
    u = g_ref[...]
    s_lo, s_hi = _unpack(u[:, :HDIM])
    e_lo, e_hi = _unpack(u[:, HDIM:])
    h = jnp.concatenate([s_lo + e_lo, s_hi + e_hi], axis=1)
    h = jnp.maximum(h, 0.0).astype(jnp.bfloat16)
    w = w_ref[...].astype(jnp.bfloat16)
    o_ref[...] = (
        jnp.dot(h, w, preferred_element_type=jnp.float32) + bias_ref[...]
    )


_MLP_BLK = 8000


def _mlp_out(g, W2, b2):
    return pl.pallas_call(
        _mlp_kernel,
        grid=(N_EDGES // _MLP_BLK,),
        in_specs=[
            pl.BlockSpec((_MLP_BLK, DIM), lambda i: (i, 0)),
            pl.BlockSpec((DIM, DIM), lambda i: (0, 0)),
            pl.BlockSpec((1, DIM), lambda i: (0, 0)),
        ],
        out_specs=pl.BlockSpec((_MLP_BLK, DIM), lambda i: (i, 0)),
        out_shape=jax.ShapeDtypeStruct((N_EDGES, DIM), jnp.float32),
    )(g, W2, b2.reshape(1, DIM))


def kernel(x, edge_index, W1, b1, W2, b2):
    x2 = x.reshape(-1, x.shape[-1])
    ei = edge_index.reshape(2, -1).astype(jnp.int32)
    table = _build_table(x2, W1, b1)
    idx = ei.T.reshape(2 * N_EDGES)
    g = _sc_gather(table, idx)
    out = _mlp_out(g, W2, b2)
    return out.reshape(1, N_EDGES, DIM)


# speedup vs baseline: 1.4425x; 1.4425x over previous
import functools

import jax
import jax.numpy as jnp
from jax import lax
from jax.experimental import pallas as pl
from jax.experimental.pallas import tpu as pltpu
from jax.experimental.pallas import tpu_sc as plsc

N_NODES = 10000
N_EDGES = 320000
DIM = 128
HDIM = 64

NC, NS = 2, 16
NW = NC * NS
E_PER_W = N_EDGES // NW
CHUNK_E = 40
CHUNK = 2 * CHUNK_E
N_PHASES = 2
E_PHASE = E_PER_W // N_PHASES
N_CHUNKS = E_PHASE // CHUNK_E
STAGE_ROWS = 1000


def _round_bf16_bits(u):
    return (u + 0x7FFF + ((u >> 16) & 1)) & jnp.uint32(0xFFFF0000)


def _pack_cols(t):
    lo = lax.bitcast_convert_type(t[:, :HDIM], jnp.uint32)
    hi = lax.bitcast_convert_type(t[:, HDIM:], jnp.uint32)
    return (_round_bf16_bits(lo) >> 16) | _round_bf16_bits(hi)


def _table_kernel(x_ref, w_ref, b_ref, t_ref):
    x = x_ref[...]
    ts = jnp.dot(x, w_ref[0], preferred_element_type=jnp.float32) + b_ref[0]
    te = jnp.dot(x, w_ref[1], preferred_element_type=jnp.float32)
    t_ref[...] = jnp.concatenate([_pack_cols(ts), _pack_cols(te)], axis=1)


def _build_table(x, W1, b1):
    wr = W1.reshape(2, DIM, DIM)
    blk = 1000
    return pl.pallas_call(
        _table_kernel,
        grid=(N_NODES // blk,),
        in_specs=[
            pl.BlockSpec((blk, DIM), lambda i: (i, 0)),
            pl.BlockSpec((2, DIM, DIM), lambda i: (0, 0, 0)),
            pl.BlockSpec((1, DIM), lambda i: (0, 0)),
        ],
        out_specs=pl.BlockSpec((blk, DIM), lambda i: (i, 0)),
        out_shape=jax.ShapeDtypeStruct((N_NODES, DIM), jnp.uint32),
    )(x, wr, b1.reshape(1, DIM))


def _sc_gather(table, idx):
    mesh = plsc.VectorSubcoreMesh(
        core_axis_name="c", subcore_axis_name="s", num_cores=NC, num_subcores=NS
    )

    @functools.partial(
        pl.kernel,
        out_type=jax.ShapeDtypeStruct((N_EDGES, DIM), jnp.uint32),
        mesh=mesh,
        scratch_types=[
            pltpu.VMEM((2 * E_PHASE,), jnp.int32),
            pltpu.VMEM((2, CHUNK, DIM), jnp.uint32),
            pltpu.VMEM((2, CHUNK_E, DIM), jnp.uint32),
            pltpu.VMEM_SHARED((N_NODES, DIM), jnp.uint32),
            pltpu.SemaphoreType.DMA,
            pltpu.SemaphoreType.DMA,
            pltpu.SemaphoreType.DMA,
            pltpu.SemaphoreType.DMA,
            pltpu.SemaphoreType.DMA,
        ],
    )
    def k(t_hbm, i_hbm, o_hbm, idx_v, rows_v, cbuf, t_sp, isem, gsem0, gsem1,
          osem0, osem1):
        cid = lax.axis_index("c")
        sid = lax.axis_index("s")
        wid = sid * NC + cid
        ebase = wid * E_PER_W
        gsems = (gsem0, gsem1)
        osems = (osem0, osem1)

        @pl.when(sid < N_NODES // STAGE_ROWS)
        def _():
            pltpu.sync_copy(
                t_hbm.at[pl.ds(sid * STAGE_ROWS, STAGE_ROWS)],
                t_sp.at[pl.ds(sid * STAGE_ROWS, STAGE_ROWS)],
            )

        plsc.subcore_barrier()

        def compact(b):
            for j in range(CHUNK_E):
                for v in range(HDIM // 16):
                    s = pl.ds(16 * v, 16)
                    h = pl.ds(HDIM + 16 * v, 16)
                    cbuf[b, j, s] = rows_v[b, 2 * j, s]
                    cbuf[b, j, h] = rows_v[b, 2 * j + 1, h]

        for p in range(N_PHASES):
            pbase = ebase + p * E_PHASE

            pltpu.async_copy(
                i_hbm.at[pl.ds(2 * pbase, 2 * E_PHASE)], idx_v, isem
            ).wait()

            def gather_copy(ci, b):
                return pltpu.make_async_copy(
                    t_sp.at[idx_v.at[pl.ds(ci * CHUNK, CHUNK)]], rows_v.at[b],
                    gsems[b],
                )

            def out_copy(ci, b):
                return pltpu.make_async_copy(
                    cbuf.at[b],
                    o_hbm.at[pl.ds(pbase + ci * CHUNK_E, CHUNK_E)],
                    osems[b],
                )

            gather_copy(0, 0).start()
            n_paired = N_CHUNKS - (N_CHUNKS % 2)

            @pl.loop(0, n_paired, step=2)
            def _(i0):
                gather_copy(i0, 0).wait()
                compact(0)
                out_copy(i0, 0).start()

                @pl.when(i0 > 0)
                def _():
                    out_copy(i0 - 1, 1).wait()

                gather_copy(i0 + 1, 1).start()

                gather_copy(i0 + 1, 1).wait()
                compact(1)
                out_copy(i0 + 1, 1).start()

                @pl.when(i0 + 2 < N_CHUNKS)
                def _():
                    out_copy(i0, 0).wait()
                    gather_copy(i0 + 2, 0).start()

            if N_CHUNKS % 2:
                gather_copy(N_CHUNKS - 1, 0).wait()
                compact(0)
                out_copy(N_CHUNKS - 1, 0).start()
                out_copy(N_CHUNKS - 2, 1).wait()
                out_copy(N_CHUNKS - 1, 0).wait()
            else:
                out_copy(N_CHUNKS - 2, 0).wait()
                out_copy(N_CHUNKS - 1, 1).wait()

    return k(table, idx)


def _unpack(u):
    lo = lax.bitcast_convert_type(u << 16, jnp.float32)
    hi = lax.bitcast_convert_type(u & jnp.uint32(0xFFFF0000), jnp.float32)
    return lo, hi


def _mlp_kernel(g_ref, w_ref, bias_ref, o_ref):
    u = g_ref[...]
    s_lo, s_hi = _unpack(u[:, :HDIM])
    e_lo, e_hi = _unpack(u[:, HDIM:])
    h = jnp.concatenate([s_lo + e_lo, s_hi + e_hi], axis=1)
    h = jnp.maximum(h, 0.0).astype(jnp.bfloat16)
    w = w_ref[...].astype(jnp.bfloat16)
    o_ref[...] = (
        jnp.dot(h, w, preferred_element_type=jnp.float32) + bias_ref[...]
    )


_MLP_BLK = 8000


def _mlp_out(g, W2, b2):
    return pl.pallas_call(
        _mlp_kernel,
        grid=(N_EDGES // _MLP_BLK,),
        in_specs=[
            pl.BlockSpec((_MLP_BLK, DIM), lambda i: (i, 0)),
            pl.BlockSpec((DIM, DIM), lambda i: (0, 0)),
            pl.BlockSpec((1, DIM), lambda i: (0, 0)),
        ],
        out_specs=pl.BlockSpec((_MLP_BLK, DIM), lambda i: (i, 0)),
        out_shape=jax.ShapeDtypeStruct((N_EDGES, DIM), jnp.float32),
    )(g, W2, b2.reshape(1, DIM))


def kernel(x, edge_index, W1, b1, W2, b2):
    x2 = x.reshape(-1, x.shape[-1])
    ei = edge_index.reshape(2, -1).astype(jnp.int32)
    table = _build_table(x2, W1, b1)
    idx = ei.T.reshape(2 * N_EDGES)
    g = _sc_gather(table, idx)
    out = _mlp_out(g, W2, b2)
    return out.reshape(1, N_EDGES, DIM)


# speedup vs baseline: 2.2349x; 1.5494x over previous
import functools

import jax
import jax.numpy as jnp
from jax import lax
from jax.experimental import pallas as pl
from jax.experimental.pallas import tpu as pltpu
from jax.experimental.pallas import tpu_sc as plsc

N_NODES = 10000
N_EDGES = 320000
DIM = 128
HDIM = 64

N_SLICES = 1
E_SLICE = N_EDGES // N_SLICES

NC, NS = 2, 16
NW = NC * NS
B_SLICE = 2 * E_SLICE
B_PER_W = B_SLICE // NW
CHUNK = 80
N_CHUNKS = B_PER_W // CHUNK
STAGE_ROWS = 1000


def _round_bf16_bits(u):
    return (u + 0x7FFF + ((u >> 16) & 1)) & jnp.uint32(0xFFFF0000)


def _pack_cols(t):
    lo = lax.bitcast_convert_type(t[:, :HDIM], jnp.uint32)
    hi = lax.bitcast_convert_type(t[:, HDIM:], jnp.uint32)
    return (_round_bf16_bits(lo) >> 16) | _round_bf16_bits(hi)


def _table_kernel(x_ref, w_ref, b_ref, t_ref):
    x = x_ref[...]
    ts = jnp.dot(x, w_ref[0], preferred_element_type=jnp.float32) + b_ref[0]
    te = jnp.dot(x, w_ref[1], preferred_element_type=jnp.float32)
    t_ref[...] = jnp.concatenate([_pack_cols(ts), _pack_cols(te)], axis=1)


def _build_table(x, W1, b1):
    wr = W1.reshape(2, DIM, DIM)
    blk = 1000
    return pl.pallas_call(
        _table_kernel,
        grid=(N_NODES // blk,),
        in_specs=[
            pl.BlockSpec((blk, DIM), lambda i: (i, 0)),
            pl.BlockSpec((2, DIM, DIM), lambda i: (0, 0, 0)),
            pl.BlockSpec((1, DIM), lambda i: (0, 0)),
        ],
        out_specs=pl.BlockSpec((blk, DIM), lambda i: (i, 0)),
        out_shape=jax.ShapeDtypeStruct((N_NODES, DIM), jnp.uint32),
    )(x, wr, b1.reshape(1, DIM))


def _sc_gather(table, idx3):
    mesh = plsc.VectorSubcoreMesh(
        core_axis_name="c", subcore_axis_name="s", num_cores=NC, num_subcores=NS
    )

    @functools.partial(
        pl.kernel,
        out_type=jax.ShapeDtypeStruct((B_SLICE, DIM), jnp.uint32),
        mesh=mesh,
        scratch_types=[
            pltpu.VMEM((B_PER_W,), jnp.int32),
            pltpu.VMEM((2, CHUNK, DIM), jnp.uint32),
            pltpu.VMEM_SHARED((N_NODES, DIM), jnp.uint32),
            pltpu.SemaphoreType.DMA,
            pltpu.SemaphoreType.DMA,
            pltpu.SemaphoreType.DMA,
            pltpu.SemaphoreType.DMA,
            pltpu.SemaphoreType.DMA,
        ],
    )
    def k(t_hbm, i_hbm, o_hbm, idx_v, rows_v, t_sp, isem, gsem0, gsem1,
          osem0, osem1):
        cid = lax.axis_index("c")
        sid = lax.axis_index("s")
        wid = sid * NC + cid
        base = wid * B_PER_W
        gsems = (gsem0, gsem1)
        osems = (osem0, osem1)

        @pl.when(sid < N_NODES // STAGE_ROWS)
        def _():
            pltpu.sync_copy(
                t_hbm.at[pl.ds(sid * STAGE_ROWS, STAGE_ROWS)],
                t_sp.at[pl.ds(sid * STAGE_ROWS, STAGE_ROWS)],
            )

        plsc.subcore_barrier()

        pltpu.async_copy(i_hbm.at[pl.ds(base, B_PER_W)], idx_v, isem).wait()

        def gather_copy(ci, b):
            return pltpu.make_async_copy(
                t_sp.at[idx_v.at[pl.ds(ci * CHUNK, CHUNK)]], rows_v.at[b],
                gsems[b],
            )

        def out_copy(ci, b):
            return pltpu.make_async_copy(
                rows_v.at[b],
                o_hbm.at[pl.ds(base + ci * CHUNK, CHUNK)],
                osems[b],
            )

        gather_copy(0, 0).start()
        n_paired = N_CHUNKS - (N_CHUNKS % 2)

        @pl.loop(0, n_paired, step=2)
        def _(i0):
            gather_copy(i0, 0).wait()
            out_copy(i0, 0).start()

            @pl.when(i0 > 0)
            def _():
                out_copy(i0 - 1, 1).wait()

            gather_copy(i0 + 1, 1).start()

            gather_copy(i0 + 1, 1).wait()
            out_copy(i0 + 1, 1).start()

            @pl.when(i0 + 2 < N_CHUNKS)
            def _():
                out_copy(i0, 0).wait()
                gather_copy(i0 + 2, 0).start()

        if N_CHUNKS % 2:
            gather_copy(N_CHUNKS - 1, 0).wait()
            out_copy(N_CHUNKS - 1, 0).start()
            out_copy(N_CHUNKS - 2, 1).wait()
            out_copy(N_CHUNKS - 1, 0).wait()
        else:
            out_copy(N_CHUNKS - 2, 0).wait()
            out_copy(N_CHUNKS - 1, 1).wait()

    return k(table, idx3)


def _unpack(u):
    lo = lax.bitcast_convert_type(u << 16, jnp.float32)
    hi = lax.bitcast_convert_type(u & jnp.uint32(0xFFFF0000), jnp.float32)
    return lo, hi


def _mlp_kernel(a_ref, b_ref, w_ref, bias_ref, *rest):
    o_ref = rest[-1]
    a_lo, a_hi = _unpack(a_ref[0][:, :HDIM])
    b_lo, b_hi = _unpack(b_ref[0][:, HDIM:])
    h = jnp.concatenate([a_lo + b_lo, a_hi + b_hi], axis=1)
    h = jnp.maximum(h, 0.0).astype(jnp.bfloat16)
    w = w_ref[...].astype(jnp.bfloat16)
    o_ref[...] = (
        jnp.dot(h, w, preferred_element_type=jnp.float32) + bias_ref[...]
    )


_MLP_BLK = 8000
_BLKS_PER_SLICE = E_SLICE // _MLP_BLK


def _mlp_out(g3, W2, b2, prev, s):
    off = s * _BLKS_PER_SLICE
    in_specs = [
        pl.BlockSpec((1, _MLP_BLK, DIM), lambda i: (0, i, 0)),
        pl.BlockSpec((1, _MLP_BLK, DIM), lambda i: (1, i, 0)),
        pl.BlockSpec((DIM, DIM), lambda i: (0, 0)),
        pl.BlockSpec((1, DIM), lambda i: (0, 0)),
    ]
    args = [g3, g3, W2, b2.reshape(1, DIM)]
    aliases = {}
    if prev is not None:
        in_specs.append(pl.BlockSpec(memory_space=pl.ANY))
        args.append(prev)
        aliases = {4: 0}
    return pl.pallas_call(
        _mlp_kernel,
        grid=(_BLKS_PER_SLICE,),
        in_specs=in_specs,
        out_specs=pl.BlockSpec((_MLP_BLK, DIM), lambda i: (i + off, 0)),
        out_shape=jax.ShapeDtypeStruct((N_EDGES, DIM), jnp.float32),
        input_output_aliases=aliases,
    )(*args)


def kernel(x, edge_index, W1, b1, W2, b2):
    x2 = x.reshape(-1, x.shape[-1])
    ei = edge_index.reshape(2, -1).astype(jnp.int32)
    table = _build_table(x2, W1, b1)
    out = None
    for s in range(N_SLICES):
        idx = lax.slice(ei, (0, s * E_SLICE), (2, (s + 1) * E_SLICE))
        g = _sc_gather(table, idx.reshape(B_SLICE))
        out = _mlp_out(g.reshape(2, E_SLICE, DIM), W2, b2, out, s)
    return out.reshape(1, N_EDGES, DIM)
